# baseline (speedup 1.0000x reference)
import functools

import jax
import jax.numpy as jnp
from jax.experimental import pallas as pl
from jax.experimental.pallas import tpu as pltpu


def _pipelined_linear(bounds, h_hbm, wt_ref, b_ref, o_hbm, hbuf, obuf,
                      insem, outsem):
    nq = len(bounds) - 1

    def in_copy(i):
        lo, hi = bounds[i], bounds[i + 1]
        return pltpu.make_async_copy(
            h_hbm.at[pl.ds(lo, hi - lo), :],
            hbuf.at[pl.ds(lo, hi - lo), :],
            insem.at[i],
        )

    def out_copy(i):
        lo, hi = bounds[i], bounds[i + 1]
        return pltpu.make_async_copy(
            obuf.at[pl.ds(lo, hi - lo), :],
            o_hbm.at[pl.ds(lo, hi - lo), :],
            outsem.at[i],
        )

    for i in range(nq):
        in_copy(i).start(priority=0 if i < nq // 2 else 1)
    order = ([i // 2 if i % 2 == 0 else nq // 2 + i // 2 for i in range(nq)]
             if nq > 1 else [0])
    for i in order:
        in_copy(i).wait()
        lo, hi = bounds[i], bounds[i + 1]
        rows = pl.ds(lo, hi - lo)
        acc = jnp.dot(hbuf[rows, :], wt_ref[:, :],
                      preferred_element_type=jnp.float32)
        obuf[rows, :] = acc + b_ref[:, :]
        out_copy(i).start(priority=0 if i < nq // 2 else 1)
    for i in range(nq):
        out_copy(i).wait()


def kernel(h, edge_index, edge_sh, edge_radial, n_atoms, W, b):
    n, d = h.shape
    if n % 8 == 0 and n >= 64:
        q = (n // 4) // 8 * 8
        half = (n // 2) // 8 * 8
        bounds = (0, q, half, half + q, n)
    else:
        bounds = (0, n)
    wt = W.T
    b2 = b.reshape(1, d)
    return pl.pallas_call(
        functools.partial(_pipelined_linear, bounds),
        in_specs=[
            pl.BlockSpec(memory_space=pl.ANY),
            pl.BlockSpec((d, d), lambda: (0, 0)),
            pl.BlockSpec((1, d), lambda: (0, 0)),
        ],
        out_specs=pl.BlockSpec(memory_space=pl.ANY),
        out_shape=jax.ShapeDtypeStruct((n, d), jnp.float32),
        scratch_shapes=[
            pltpu.VMEM((n, d), jnp.float32),
            pltpu.VMEM((n, d), jnp.float32),
            pltpu.SemaphoreType.DMA((len(bounds) - 1,)),
            pltpu.SemaphoreType.DMA((len(bounds) - 1,)),
        ],
    )(h, wt, b2)
